# Initial kernel scaffold; baseline (speedup 1.0000x reference)
import jax
import jax.numpy as jnp
from jax.experimental import pallas as pl


def kernel(token_embeddings, pos_table):
    raise NotImplementedError("write your pallas kernel here")


# speedup vs baseline: 1.4870x; 1.4870x over previous
import jax
import jax.numpy as jnp
from jax.experimental import pallas as pl
from jax.experimental.pallas import tpu as pltpu

_BS = 512


def _add_body(tok_ref, pos_ref, out_ref):
    out_ref[...] = tok_ref[...] + pos_ref[...]


def kernel(token_embeddings, pos_table):
    batch, seq, dim = token_embeddings.shape
    grid = (seq // _BS, batch)
    return pl.pallas_call(
        _add_body,
        grid=grid,
        in_specs=[
            pl.BlockSpec((1, _BS, dim), lambda s, b: (b, s, 0)),
            pl.BlockSpec((_BS, dim), lambda s, b: (s, 0)),
        ],
        out_specs=pl.BlockSpec((1, _BS, dim), lambda s, b: (b, s, 0)),
        out_shape=jax.ShapeDtypeStruct((batch, seq, dim), token_embeddings.dtype),
    )(token_embeddings, pos_table)
